# baseline (speedup 1.0000x reference)
import functools

import jax
import jax.numpy as jnp
from jax import lax
from jax.experimental import pallas as pl
from jax.experimental.pallas import tpu as pltpu
from jax.experimental.pallas import tpu_sc as plsc

D = 128
N_USERS = 50000
N_ITEMS = 50000
N_UFEAT = 3207
N_IFEAT = 2094
B = 1024
FANOUT = 16
FEAT = 8

NC, NS = 2, 16
NW = NC * NS

GROWS = 128
QCOLS = 32
NQ = D // QCOLS
NG = NW // NQ

U_PAD = 51200
UG = U_PAD // NG
CU_A = 128
NCH_A = UG // CU_A
NPAIR_A = NCH_A // 2

E2 = B * FANOUT * FANOUT
E2W = E2 // NW
CE_B = 256
NCH_B = E2W // CE_B
NPAIR_B = NCH_B // 2
GPC_B = CE_B // FANOUT

E1 = B * FANOUT
EG_C = E1 // NG
CE_C = 128
NCH_C = EG_C // CE_C
NPAIR_C = NCH_C // 2

RMID = 2048


def _worker_id():
    return lax.axis_index("s") * NC + lax.axis_index("c")


def _mean_rows(rows_v, acc_v, n_out, group, scale, out_base=0):
    def per_g(g, carry):
        for dd in range(D // 16):
            sl = pl.ds(dd * 16, 16)
            acc = rows_v[g * group, sl]
            for f in range(1, group):
                acc = acc + rows_v[g * group + f, sl]
            acc_v[out_base + g, sl] = acc * scale
        return carry
    lax.fori_loop(0, n_out, per_g, 0)


def _fire_gather(table_hbm, idx_v, idx_off, rows_v, n_rows, sem):
    for h in range(n_rows // GROWS):
        pltpu.async_copy(
            table_hbm.at[idx_v.at[pl.ds(idx_off + h * GROWS, GROWS)]],
            rows_v.at[pl.ds(h * GROWS, GROWS)], sem)


def _wait_gather(table_hbm, idx_v, rows_v, n_rows, sem):
    for h in range(n_rows // GROWS):
        pltpu.make_async_copy(
            table_hbm.at[idx_v.at[pl.ds(0, GROWS)]],
            rows_v.at[pl.ds(h * GROWS, GROWS)], sem).wait()


@functools.lru_cache(maxsize=None)
def _build_sc_kernels():
    mesh = plsc.VectorSubcoreMesh(core_axis_name="c", subcore_axis_name="s")
    iota16 = lambda: lax.iota(jnp.int32, 16)

    @functools.partial(
        pl.kernel, mesh=mesh,
        compiler_params=pltpu.CompilerParams(use_tc_tiling_on_sc=False, needs_layout_passes=False),
        out_type=jax.ShapeDtypeStruct((U_PAD, D), jnp.float32),
        scratch_types=[
            pltpu.VMEM((N_UFEAT, QCOLS), jnp.float32),
            pltpu.VMEM((FEAT, CU_A), jnp.int32),
            pltpu.VMEM((FEAT, CU_A), jnp.int32),
            pltpu.VMEM((CU_A, QCOLS), jnp.float32),
            pltpu.VMEM((CU_A, QCOLS), jnp.float32),
            pltpu.SemaphoreType.DMA,
            pltpu.SemaphoreType.DMA,
            pltpu.SemaphoreType.DMA,
            pltpu.SemaphoreType.DMA,
        ],
    )
    def ufm_kernel(ufit_hbm, emb_hbm, out_hbm, table_v, idx0, idx1,
                   out0, out1, isem0, isem1, wsem0, wsem1):
        wid = _worker_id()
        qid = wid % NQ
        gid = wid // NQ
        c0 = qid * QCOLS
        ub = gid * UG
        pltpu.sync_copy(emb_hbm.at[:, pl.ds(c0, QCOLS)], table_v)

        def fire_idx(i, idxv, sem):
            pltpu.async_copy(
                ufit_hbm.at[:, pl.ds(ub + i * CU_A, CU_A)], idxv, sem)

        def compute(idxv, outv):
            def per_g(g, carry):
                b = g * 16
                rowsel = iota16() + b
                vf = [idxv[f, pl.ds(b, 16)] for f in range(FEAT)]
                for c in range(QCOLS):
                    cc = jnp.full((16,), c, jnp.int32)
                    acc = plsc.load_gather(table_v, [vf[0], cc])
                    for f in range(1, FEAT):
                        acc = acc + plsc.load_gather(table_v, [vf[f], cc])
                    plsc.store_scatter(outv, [rowsel, cc], acc * (1.0 / FEAT))
                return carry
            lax.fori_loop(0, CU_A // 16, per_g, 0)

        def phase(j, i, idxv, isem, outv, wsem):
            pltpu.make_async_copy(
                ufit_hbm.at[:, pl.ds(ub, CU_A)], idxv, isem).wait()

            @pl.when(j > 0)
            def _():
                pltpu.make_async_copy(
                    outv, out_hbm.at[pl.ds(ub, CU_A), pl.ds(c0, QCOLS)],
                    wsem).wait()
            compute(idxv, outv)
            pltpu.async_copy(
                outv,
                out_hbm.at[pl.ds(ub + i * CU_A, CU_A), pl.ds(c0, QCOLS)],
                wsem)

        def pair(j, carry):
            i0 = 2 * j
            fire_idx(i0 + 1, idx1, isem1)
            phase(j, i0, idx0, isem0, out0, wsem0)

            @pl.when(j < NPAIR_A - 1)
            def _():
                fire_idx(i0 + 2, idx0, isem0)
            phase(j, i0 + 1, idx1, isem1, out1, wsem1)
            return carry

        fire_idx(0, idx0, isem0)
        lax.fori_loop(0, NPAIR_A, pair, 0)
        pltpu.make_async_copy(
            out0, out_hbm.at[pl.ds(ub, CU_A), pl.ds(c0, QCOLS)], wsem0).wait()
        pltpu.make_async_copy(
            out1, out_hbm.at[pl.ds(ub, CU_A), pl.ds(c0, QCOLS)], wsem1).wait()

    @functools.partial(
        pl.kernel, mesh=mesh,
        compiler_params=pltpu.CompilerParams(use_tc_tiling_on_sc=False, needs_layout_passes=False),
        out_type=(jax.ShapeDtypeStruct((B, D), jnp.float32),
                  jax.ShapeDtypeStruct((E1, D), jnp.float32)),
        scratch_types=[
            pltpu.VMEM((E2W,), jnp.int32),
            pltpu.VMEM((CE_B, D), jnp.float32),
            pltpu.VMEM((CE_B, D), jnp.float32),
            pltpu.VMEM((GPC_B, D), jnp.float32),
            pltpu.VMEM((GPC_B, D), jnp.float32),
            pltpu.VMEM((B // NW,), jnp.int32),
            pltpu.VMEM((B // NW, D), jnp.float32),
            pltpu.SemaphoreType.DMA,
            pltpu.SemaphoreType.DMA,
            pltpu.SemaphoreType.DMA,
            pltpu.SemaphoreType.DMA,
        ],
    )
    def l2_kernel(ufm_hbm, n0_hbm, n2_hbm, h0_hbm, m2_hbm,
                  idx_v, rows0, rows1, acc0, acc1, nbr0_v, rows0b_v,
                  sem0, sem1, wsem0, wsem1):
        wid = _worker_id()
        r0 = wid * (B // NW)
        pltpu.sync_copy(n0_hbm.at[pl.ds(r0, B // NW)], nbr0_v)
        pltpu.async_copy(ufm_hbm.at[nbr0_v], rows0b_v, sem0).wait()
        pltpu.sync_copy(rows0b_v, h0_hbm.at[pl.ds(r0, B // NW)])
        eb = wid * E2W
        gb = wid * (E2W // FANOUT)
        pltpu.sync_copy(n2_hbm.at[pl.ds(eb, E2W)], idx_v)

        def phase(j, i, rows, sem, acc, wsem):
            _wait_gather(ufm_hbm, idx_v, rows, CE_B, sem)

            @pl.when(j > 0)
            def _():
                pltpu.make_async_copy(
                    acc, m2_hbm.at[pl.ds(gb, GPC_B)], wsem).wait()
            _mean_rows(rows, acc, GPC_B, FANOUT, 1.0 / FANOUT)
            pltpu.async_copy(acc, m2_hbm.at[pl.ds(gb + i * GPC_B, GPC_B)],
                             wsem)

        def pair(j, carry):
            i0 = 2 * j
            _fire_gather(ufm_hbm, idx_v, (i0 + 1) * CE_B, rows1, CE_B, sem1)
            phase(j, i0, rows0, sem0, acc0, wsem0)

            @pl.when(j < NPAIR_B - 1)
            def _():
                _fire_gather(ufm_hbm, idx_v, (i0 + 2) * CE_B, rows0, CE_B,
                             sem0)
            phase(j, i0 + 1, rows1, sem1, acc1, wsem1)
            return carry

        _fire_gather(ufm_hbm, idx_v, 0, rows0, CE_B, sem0)
        lax.fori_loop(0, NPAIR_B, pair, 0)
        pltpu.make_async_copy(acc0, m2_hbm.at[pl.ds(gb, GPC_B)], wsem0).wait()
        pltpu.make_async_copy(acc1, m2_hbm.at[pl.ds(gb, GPC_B)], wsem1).wait()

    @functools.partial(
        pl.kernel, mesh=mesh,
        compiler_params=pltpu.CompilerParams(use_tc_tiling_on_sc=False, needs_layout_passes=False),
        out_type=(jax.ShapeDtypeStruct((E1, D), jnp.float32),
                  jax.ShapeDtypeStruct((B, D), jnp.float32)),
        scratch_types=[
            pltpu.VMEM((N_IFEAT, QCOLS), jnp.float32),
            pltpu.VMEM((EG_C,), jnp.int32),
            pltpu.VMEM((CE_C, FEAT), jnp.int32),
            pltpu.VMEM((CE_C, FEAT), jnp.int32),
            pltpu.VMEM((CE_C, QCOLS), jnp.float32),
            pltpu.VMEM((CE_C, QCOLS), jnp.float32),
            pltpu.VMEM((B // NG, QCOLS), jnp.float32),
            pltpu.SemaphoreType.DMA,
            pltpu.SemaphoreType.DMA,
            pltpu.SemaphoreType.DMA,
            pltpu.SemaphoreType.DMA,
        ],
    )
    def item_kernel(n1_hbm, ifi2_hbm, emb_hbm, h1_hbm, m1_hbm,
                    table_v, nbr_v, idx80, idx81, out0, out1, m1_v,
                    gsem0, gsem1, wsem0, wsem1):
        wid = _worker_id()
        qid = wid % NQ
        gid = wid // NQ
        c0 = qid * QCOLS
        eb = gid * EG_C
        pltpu.sync_copy(emb_hbm.at[:, pl.ds(c0, QCOLS)], table_v)
        pltpu.sync_copy(n1_hbm.at[pl.ds(eb, EG_C)], nbr_v)

        def fire_idx8(i, idx8v, sem):
            pltpu.async_copy(
                ifi2_hbm.at[nbr_v.at[pl.ds(i * CE_C, CE_C)]], idx8v, sem)

        def compute(i, idx8v, outv):
            def per_g(g, carry):
                b = g * 16
                rowsel = iota16() + b
                vf = [plsc.load_gather(idx8v,
                                       [rowsel, jnp.full((16,), f, jnp.int32)])
                      for f in range(FEAT)]
                for c in range(QCOLS):
                    cc = jnp.full((16,), c, jnp.int32)
                    acc = plsc.load_gather(table_v, [vf[0], cc])
                    for f in range(1, FEAT):
                        acc = acc + plsc.load_gather(table_v, [vf[f], cc])
                    plsc.store_scatter(outv, [rowsel, cc], acc * (1.0 / FEAT))
                return carry
            lax.fori_loop(0, CE_C // 16, per_g, 0)
            def per_m(g, carry):
                for s in range(QCOLS // 16):
                    sl = pl.ds(s * 16, 16)
                    acc = outv[g * FANOUT, sl]
                    for f in range(1, FANOUT):
                        acc = acc + outv[g * FANOUT + f, sl]
                    m1_v[i * (CE_C // FANOUT) + g, sl] = acc * (1.0 / FANOUT)
                return carry
            lax.fori_loop(0, CE_C // FANOUT, per_m, 0)

        def phase(j, i, idx8v, gsem, outv, wsem):
            pltpu.make_async_copy(
                ifi2_hbm.at[nbr_v.at[pl.ds(0, CE_C)]], idx8v, gsem).wait()

            @pl.when(j > 0)
            def _():
                pltpu.make_async_copy(
                    outv, h1_hbm.at[pl.ds(eb, CE_C), pl.ds(c0, QCOLS)],
                    wsem).wait()
            compute(i, idx8v, outv)
            pltpu.async_copy(
                outv,
                h1_hbm.at[pl.ds(eb + i * CE_C, CE_C), pl.ds(c0, QCOLS)],
                wsem)

        def pair(j, carry):
            i0 = 2 * j
            fire_idx8(i0 + 1, idx81, gsem1)
            phase(j, i0, idx80, gsem0, out0, wsem0)

            @pl.when(j < NPAIR_C - 1)
            def _():
                fire_idx8(i0 + 2, idx80, gsem0)
            phase(j, i0 + 1, idx81, gsem1, out1, wsem1)
            return carry

        fire_idx8(0, idx80, gsem0)
        lax.fori_loop(0, NPAIR_C, pair, 0)
        pltpu.make_async_copy(
            out0, h1_hbm.at[pl.ds(eb, CE_C), pl.ds(c0, QCOLS)], wsem0).wait()
        pltpu.make_async_copy(
            out1, h1_hbm.at[pl.ds(eb, CE_C), pl.ds(c0, QCOLS)], wsem1).wait()
        pltpu.sync_copy(
            m1_v, m1_hbm.at[pl.ds(gid * (B // NG), B // NG),
                            pl.ds(c0, QCOLS)])

    return ufm_kernel, l2_kernel, item_kernel


def _mid_body(h1r, m2r, Wi, bi, Wu, bu, W0a, W0b, b0, P, mm1):
    h1 = jnp.dot(h1r[...], Wi[...], preferred_element_type=jnp.float32) + bi[...]
    m2 = jnp.dot(m2r[...], Wu[...], preferred_element_type=jnp.float32) + bu[...]
    nh1 = jnp.maximum(
        jnp.dot(h1, W0a[...], preferred_element_type=jnp.float32)
        + jnp.dot(m2, W0b[...], preferred_element_type=jnp.float32)
        + b0[...], 0.0)
    mm1[...] = jnp.dot(P[...], nh1, preferred_element_type=jnp.float32)


def _head_body(h0r, m1r, mm1, Wu, bu, Wi, bi, W0a, W0b, b0, W1a, W1b, b1, out):
    h0 = jnp.dot(h0r[...], Wu[...], preferred_element_type=jnp.float32) + bu[...]
    m1 = jnp.dot(m1r[...], Wi[...], preferred_element_type=jnp.float32) + bi[...]
    nh0 = jnp.maximum(
        jnp.dot(h0, W0a[...], preferred_element_type=jnp.float32)
        + jnp.dot(m1, W0b[...], preferred_element_type=jnp.float32)
        + b0[...], 0.0)
    out[...] = (jnp.dot(nh0, W1a[...], preferred_element_type=jnp.float32)
                + jnp.dot(mm1[...], W1b[...], preferred_element_type=jnp.float32)
                + b1[...])


def kernel(neighbors_l0, neighbors_l1, neighbors_l2, offsets_l1, offsets_l2,
           user_feature_indices, user_feature_offsets, item_feature_indices,
           item_feature_offsets, user_feature_emb, item_feature_emb,
           user_proj_W, user_proj_b, item_proj_W, item_proj_b,
           w0_W, w0_b, w1_W, w1_b):
    n0 = neighbors_l0.astype(jnp.int32)
    n1 = neighbors_l1.astype(jnp.int32)
    n2 = neighbors_l2.astype(jnp.int32)
    ufi = user_feature_indices.astype(jnp.int32)
    ifi = item_feature_indices.astype(jnp.int32)

    ufi_t = jnp.pad(ufi, (0, (U_PAD - N_USERS) * FEAT)).reshape(U_PAD, FEAT).T
    ifi2 = ifi.reshape(N_ITEMS, FEAT)

    ufm_kernel, l2_kernel, item_kernel = _build_sc_kernels()
    ufm = ufm_kernel(ufi_t, user_feature_emb)
    h0_raw, m2_raw = l2_kernel(ufm, n0, n2)
    h1_raw, m1_raw = item_kernel(n1, ifi2, item_feature_emb)

    W0a, W0b = w0_W[:D], w0_W[D:]
    W1a, W1b = w1_W[:D], w1_W[D:]
    bu2, bi2 = user_proj_b[None, :], item_proj_b[None, :]
    b02, b12 = w0_b[None, :], w1_b[None, :]
    pool = jnp.kron(jnp.eye(RMID // FANOUT, dtype=jnp.float32),
                    jnp.full((1, FANOUT), 1.0 / FANOUT, dtype=jnp.float32))

    full = lambda s: pl.BlockSpec(s, lambda i: (0, 0))
    mm1 = pl.pallas_call(
        _mid_body,
        grid=(E1 // RMID,),
        in_specs=[
            pl.BlockSpec((RMID, D), lambda i: (i, 0)),
            pl.BlockSpec((RMID, D), lambda i: (i, 0)),
            full((D, D)), full((1, D)), full((D, D)), full((1, D)),
            full((D, D)), full((D, D)), full((1, D)),
            full((RMID // FANOUT, RMID)),
        ],
        out_specs=pl.BlockSpec((RMID // FANOUT, D), lambda i: (i, 0)),
        out_shape=jax.ShapeDtypeStruct((B, D), jnp.float32),
    )(h1_raw, m2_raw, item_proj_W, bi2, user_proj_W, bu2, W0a, W0b, b02, pool)

    out = pl.pallas_call(
        _head_body,
        out_shape=jax.ShapeDtypeStruct((B, D), jnp.float32),
    )(h0_raw, m1_raw, mm1, user_proj_W, bu2, item_proj_W, bi2,
      W0a, W0b, b02, W1a, W1b, b12)
    return out


# speedup vs baseline: 6.2314x; 6.2314x over previous
import functools

import jax
import jax.numpy as jnp
from jax import lax
from jax.experimental import pallas as pl
from jax.experimental.pallas import tpu as pltpu
from jax.experimental.pallas import tpu_sc as plsc

D = 128
N_USERS = 50000
N_ITEMS = 50000
N_UFEAT = 3207
N_IFEAT = 2094
B = 1024
FANOUT = 16
FEAT = 8

NC, NS = 2, 16
NW = NC * NS

GROWS = 128
QCOLS = 32
NQ = D // QCOLS
NG = NW // NQ

U_PAD = 51200
AQ = 2
AG = NW // AQ
ACOLS = D // AQ
UG = U_PAD // AG
CU_A = 80
NCH_A = UG // CU_A

E2 = B * FANOUT * FANOUT
E2W = E2 // NW
CE_B = 128
NCH_B = E2W // CE_B
GPC_B = CE_B // FANOUT

E1 = B * FANOUT
EG_C = E1 // AG
CE_C = 64
NCH_C = EG_C // CE_C

RMID = 2048


def _worker_id():
    return lax.axis_index("s") * NC + lax.axis_index("c")


def _mean_rows(rows_v, acc_v, n_out, group, scale, out_base=0):
    def per_g(g, carry):
        base = g * group
        for dd in range(D // 16):
            sl = pl.ds(dd * 16, 16)
            t = [rows_v[base + 2 * k, sl] + rows_v[base + 2 * k + 1, sl]
                 for k in range(group // 2)]
            while len(t) > 1:
                t = [t[2 * k] + t[2 * k + 1] for k in range(len(t) // 2)]
            acc_v[out_base + g, sl] = t[0] * scale
        return carry
    lax.fori_loop(0, n_out, per_g, 0)


def _mean_rows_bf(rows_v, acc_v, n_out, group, scale, out_base=0):
    def per_g(g, carry):
        base = g * group
        for dd in range(D // 32):
            sl = pl.ds(dd * 32, 32)
            t = [rows_v[base + 2 * k, sl] + rows_v[base + 2 * k + 1, sl]
                 for k in range(group // 2)]
            while len(t) > 1:
                t = [t[2 * k] + t[2 * k + 1] for k in range(len(t) // 2)]
            acc_v[out_base + g, sl] = t[0] * scale
        return carry
    lax.fori_loop(0, n_out, per_g, 0)


def _fire_gather(table_hbm, idx_v, idx_off, rows_v, n_rows, sem):
    for h in range(n_rows // GROWS):
        pltpu.async_copy(
            table_hbm.at[idx_v.at[pl.ds(idx_off + h * GROWS, GROWS)]],
            rows_v.at[pl.ds(h * GROWS, GROWS)], sem)


def _wait_gather(table_hbm, idx_v, rows_v, n_rows, sem):
    for h in range(n_rows // GROWS):
        pltpu.make_async_copy(
            table_hbm.at[idx_v.at[pl.ds(0, GROWS)]],
            rows_v.at[pl.ds(h * GROWS, GROWS)], sem).wait()


@functools.lru_cache(maxsize=None)
def _build_sc_kernels():
    mesh = plsc.VectorSubcoreMesh(core_axis_name="c", subcore_axis_name="s")
    params = pltpu.CompilerParams(use_tc_tiling_on_sc=False,
                                  needs_layout_passes=False)

    @functools.partial(
        pl.kernel, mesh=mesh, compiler_params=params,
        out_type=jax.ShapeDtypeStruct((U_PAD, D), jnp.bfloat16),
        scratch_types=(
            [pltpu.VMEM((N_UFEAT, ACOLS), jnp.bfloat16)]
            + [pltpu.VMEM((CU_A * FEAT,), jnp.int32)] * 4
            + [pltpu.VMEM((CU_A, ACOLS), jnp.bfloat16)] * 4
            + [pltpu.SemaphoreType.DMA] * 8
        ),
    )
    def ufm_kernel(ufi_hbm, emb_hbm, out_hbm, table_v,
                   ix0, ix1, ix2, ix3, ot0, ot1, ot2, ot3,
                   is0, is1, is2, is3, ws0, ws1, ws2, ws3):
        ixs, ots = [ix0, ix1, ix2, ix3], [ot0, ot1, ot2, ot3]
        isems, wsems = [is0, is1, is2, is3], [ws0, ws1, ws2, ws3]
        wid = _worker_id()
        qid = wid % AQ
        gid = wid // AQ
        c0 = qid * ACOLS
        ub = gid * UG
        pltpu.sync_copy(emb_hbm.at[:, pl.ds(c0, ACOLS)], table_v)

        def fire_idx(i, idxv, sem):
            pltpu.async_copy(
                ufi_hbm.at[pl.ds((ub + i * CU_A) * FEAT, CU_A * FEAT)],
                idxv, sem)

        def compute(idxs, outv):
            def per_blk(pb, carry):
                vs = [idxs[pl.ds((pb * 4 + q) * 16, 16)] for q in range(4)]
                for q in range(4):
                    for half in range(2):
                        u = (pb * 4 + q) * 2 + half
                        r = [vs[q][half * FEAT + f] for f in range(FEAT)]
                        for s in range(ACOLS // 32):
                            sl = pl.ds(s * 32, 32)
                            t01 = table_v[r[0], sl] + table_v[r[1], sl]
                            t23 = table_v[r[2], sl] + table_v[r[3], sl]
                            t45 = table_v[r[4], sl] + table_v[r[5], sl]
                            t67 = table_v[r[6], sl] + table_v[r[7], sl]
                            outv[u, sl] = (((t01 + t23) + (t45 + t67))
                                           * (1.0 / FEAT))
                return carry
            lax.fori_loop(0, CU_A // 8, per_blk, 0)

        def quad(j, carry):
            i0 = 4 * j
            for r in range(4):
                i = i0 + r
                pltpu.make_async_copy(
                    ufi_hbm.at[pl.ds(ub * FEAT, CU_A * FEAT)],
                    ixs[r], isems[r]).wait()

                @pl.when(j > 0)
                def _():
                    pltpu.make_async_copy(
                        ots[r], out_hbm.at[pl.ds(ub, CU_A), pl.ds(c0, ACOLS)],
                        wsems[r]).wait()
                compute(ixs[r], ots[r])
                pltpu.async_copy(
                    ots[r],
                    out_hbm.at[pl.ds(ub + i * CU_A, CU_A), pl.ds(c0, ACOLS)],
                    wsems[r])

                @pl.when(i + 4 < NCH_A)
                def _():
                    fire_idx(i + 4, ixs[r], isems[r])
            return carry

        for r in range(4):
            fire_idx(r, ixs[r], isems[r])
        lax.fori_loop(0, NCH_A // 4, quad, 0)
        for r in range(4):
            pltpu.make_async_copy(
                ots[r], out_hbm.at[pl.ds(ub, CU_A), pl.ds(c0, ACOLS)],
                wsems[r]).wait()

    @functools.partial(
        pl.kernel, mesh=mesh, compiler_params=params,
        out_type=(jax.ShapeDtypeStruct((B, D), jnp.bfloat16),
                  jax.ShapeDtypeStruct((E1, D), jnp.bfloat16),
                  jax.ShapeDtypeStruct((E1, D), jnp.bfloat16),
                  jax.ShapeDtypeStruct((B, D), jnp.bfloat16)),
        scratch_types=(
            [pltpu.VMEM((E2W,), jnp.int32)]
            + [pltpu.VMEM((CE_B, D), jnp.bfloat16)] * 2
            + [pltpu.VMEM((GPC_B, D), jnp.bfloat16)] * 2
            + [pltpu.VMEM((B // NW,), jnp.int32),
               pltpu.VMEM((B // NW, D), jnp.bfloat16)]
            + [pltpu.VMEM((N_IFEAT, ACOLS), jnp.bfloat16),
               pltpu.VMEM((EG_C,), jnp.int32),
               pltpu.VMEM((EG_C, 2 * FEAT), jnp.int32)]
            + [pltpu.VMEM((CE_C, ACOLS), jnp.bfloat16)] * 4
            + [pltpu.VMEM((B // AG, ACOLS), jnp.bfloat16)]
            + [pltpu.SemaphoreType.DMA] * 9
        ),
    )
    def agg_kernel(ufm_hbm, n0_hbm, n2_hbm, n1_hbm, ifi16_hbm, iemb_hbm,
                   h0_hbm, m2_hbm, h1_hbm, m1_hbm,
                   idx_v, rw0, rw1, ac0, ac1, nbr0_v, rows0b_v,
                   ctab_v, cnbr_v, idx16_v, ot0, ot1, ot2, ot3, m1_v,
                   bg0, bg1, bw0, bw1, cg, cw0, cw1, cw2, cw3):
        rws, acs = [rw0, rw1], [ac0, ac1]
        bgsems, bwsems = [bg0, bg1], [bw0, bw1]
        ots = [ot0, ot1, ot2, ot3]
        cwsems = [cw0, cw1, cw2, cw3]
        wid = _worker_id()
        qid = wid % AQ
        gid = wid // AQ
        c0 = qid * ACOLS
        ceb = gid * EG_C
        pltpu.sync_copy(n1_hbm.at[pl.ds(ceb, EG_C)], cnbr_v)
        for i in range(EG_C // GROWS):
            pltpu.async_copy(
                ifi16_hbm.at[cnbr_v.at[pl.ds(i * GROWS, GROWS)]],
                idx16_v.at[pl.ds(i * GROWS, GROWS)], cg)
        pltpu.async_copy(iemb_hbm.at[:, pl.ds(c0, ACOLS)], ctab_v, cg)
        r0 = wid * (B // NW)
        pltpu.sync_copy(n0_hbm.at[pl.ds(r0, B // NW)], nbr0_v)
        pltpu.async_copy(ufm_hbm.at[nbr0_v], rows0b_v, bw0)
        eb = wid * E2W
        gb = wid * (E2W // FANOUT)
        pltpu.sync_copy(n2_hbm.at[pl.ds(eb, E2W)], idx_v)

        def bfire(i, rows, sem):
            pltpu.async_copy(
                ufm_hbm.at[idx_v.at[pl.ds(i * CE_B, CE_B)]], rows, sem)

        for r in range(2):
            bfire(r, rws[r], bgsems[r])
        pltpu.make_async_copy(ufm_hbm.at[nbr0_v], rows0b_v, bw0).wait()
        pltpu.sync_copy(rows0b_v, h0_hbm.at[pl.ds(r0, B // NW)])

        def bpair(j, carry):
            i0 = 2 * j
            for r in range(2):
                i = i0 + r
                pltpu.make_async_copy(
                    ufm_hbm.at[idx_v.at[pl.ds(0, CE_B)]], rws[r],
                    bgsems[r]).wait()

                @pl.when(j > 0)
                def _():
                    pltpu.make_async_copy(
                        acs[r], m2_hbm.at[pl.ds(gb, GPC_B)], bwsems[r]).wait()
                _mean_rows_bf(rws[r], acs[r], GPC_B, FANOUT, 1.0 / FANOUT)
                pltpu.async_copy(
                    acs[r], m2_hbm.at[pl.ds(gb + i * GPC_B, GPC_B)],
                    bwsems[r])

                @pl.when(i + 2 < NCH_B)
                def _():
                    bfire(i + 2, rws[r], bgsems[r])
            return carry

        lax.fori_loop(0, NCH_B // 2, bpair, 0)
        for r in range(2):
            pltpu.make_async_copy(
                acs[r], m2_hbm.at[pl.ds(gb, GPC_B)], bwsems[r]).wait()

        for i in range(EG_C // GROWS):
            pltpu.make_async_copy(
                ifi16_hbm.at[cnbr_v.at[pl.ds(0, GROWS)]],
                idx16_v.at[pl.ds(i * GROWS, GROWS)], cg).wait()
        pltpu.make_async_copy(
            iemb_hbm.at[:, pl.ds(c0, ACOLS)], ctab_v, cg).wait()

        def compute(i, outv):
            def per_blk(eb8, carry):
                vs = [idx16_v[i * CE_C + eb8 * 8 + k] for k in range(8)]
                for k in range(8):
                    e = eb8 * 8 + k
                    r = [vs[k][f] for f in range(FEAT)]
                    for s in range(ACOLS // 32):
                        sl = pl.ds(s * 32, 32)
                        t01 = ctab_v[r[0], sl] + ctab_v[r[1], sl]
                        t23 = ctab_v[r[2], sl] + ctab_v[r[3], sl]
                        t45 = ctab_v[r[4], sl] + ctab_v[r[5], sl]
                        t67 = ctab_v[r[6], sl] + ctab_v[r[7], sl]
                        outv[e, sl] = ((t01 + t23) + (t45 + t67)) * (1.0 / FEAT)
                return carry
            lax.fori_loop(0, CE_C // 8, per_blk, 0)
            def per_m(g, carry):
                for s in range(ACOLS // 32):
                    sl = pl.ds(s * 32, 32)
                    t = [outv[g * FANOUT + f, sl] + outv[g * FANOUT + f + 8, sl]
                         for f in range(8)]
                    acc = ((t[0] + t[1]) + (t[2] + t[3])) \
                        + ((t[4] + t[5]) + (t[6] + t[7]))
                    m1_v[i * (CE_C // FANOUT) + g, sl] = acc * (1.0 / FANOUT)
                return carry
            lax.fori_loop(0, CE_C // FANOUT, per_m, 0)

        def cquad(j, carry):
            i0 = 4 * j
            for r in range(4):
                i = i0 + r

                @pl.when(j > 0)
                def _():
                    pltpu.make_async_copy(
                        ots[r], h1_hbm.at[pl.ds(ceb, CE_C), pl.ds(c0, ACOLS)],
                        cwsems[r]).wait()
                compute(i, ots[r])
                pltpu.async_copy(
                    ots[r],
                    h1_hbm.at[pl.ds(ceb + i * CE_C, CE_C), pl.ds(c0, ACOLS)],
                    cwsems[r])
            return carry

        lax.fori_loop(0, NCH_C // 4, cquad, 0)
        for r in range(4):
            pltpu.make_async_copy(
                ots[r], h1_hbm.at[pl.ds(ceb, CE_C), pl.ds(c0, ACOLS)],
                cwsems[r]).wait()
        pltpu.sync_copy(
            m1_v, m1_hbm.at[pl.ds(gid * (B // AG), B // AG),
                            pl.ds(c0, ACOLS)])

    return ufm_kernel, agg_kernel


def _tc_body(h1r, m2r, P1, P2, c0v, pool, h0r, m1r, P3, P4, c0h,
             W1a, W1b, b1, out, mm1_acc):
    i = pl.program_id(0)
    nh1 = jnp.maximum(
        jnp.dot(h1r[...].astype(jnp.float32), P1[...],
                preferred_element_type=jnp.float32)
        + jnp.dot(m2r[...].astype(jnp.float32), P2[...],
                  preferred_element_type=jnp.float32)
        + c0v[...], 0.0)
    mm1_acc[pl.ds(i * (RMID // FANOUT), RMID // FANOUT), :] = jnp.dot(
        pool[...], nh1, preferred_element_type=jnp.float32)

    @pl.when(i == E1 // RMID - 1)
    def _():
        nh0 = jnp.maximum(
            jnp.dot(h0r[...].astype(jnp.float32), P3[...],
                    preferred_element_type=jnp.float32)
            + jnp.dot(m1r[...].astype(jnp.float32), P4[...],
                      preferred_element_type=jnp.float32)
            + c0h[...], 0.0)
        out[...] = (jnp.dot(nh0, W1a[...], preferred_element_type=jnp.float32)
                    + jnp.dot(mm1_acc[...], W1b[...],
                              preferred_element_type=jnp.float32)
                    + b1[...])


def kernel(neighbors_l0, neighbors_l1, neighbors_l2, offsets_l1, offsets_l2,
           user_feature_indices, user_feature_offsets, item_feature_indices,
           item_feature_offsets, user_feature_emb, item_feature_emb,
           user_proj_W, user_proj_b, item_proj_W, item_proj_b,
           w0_W, w0_b, w1_W, w1_b):
    n0 = neighbors_l0.astype(jnp.int32)
    n1 = neighbors_l1.astype(jnp.int32)
    n2 = neighbors_l2.astype(jnp.int32)
    ufi = user_feature_indices.astype(jnp.int32)
    ifi = item_feature_indices.astype(jnp.int32)

    ufi_pad = jnp.pad(ufi, (0, (U_PAD - N_USERS) * FEAT))
    ifi16 = jnp.tile(ifi.reshape(N_ITEMS, FEAT), (1, 2))

    ufm_kernel, agg_kernel = _build_sc_kernels()
    ufm = ufm_kernel(ufi_pad, user_feature_emb.astype(jnp.bfloat16))
    h0_raw, m2_raw, h1_raw, m1_raw = agg_kernel(
        ufm, n0, n2, n1, ifi16, item_feature_emb.astype(jnp.bfloat16))

    W0a, W0b = w0_W[:D], w0_W[D:]
    W1a, W1b = w1_W[:D], w1_W[D:]
    P1 = item_proj_W @ W0a
    P2 = user_proj_W @ W0b
    P3 = user_proj_W @ W0a
    P4 = item_proj_W @ W0b
    c0v = (item_proj_b @ W0a + user_proj_b @ W0b + w0_b)[None, :]
    c0h = (user_proj_b @ W0a + item_proj_b @ W0b + w0_b)[None, :]
    b12 = w1_b[None, :]
    pool = jnp.kron(jnp.eye(RMID // FANOUT, dtype=jnp.float32),
                    jnp.full((1, FANOUT), 1.0 / FANOUT, dtype=jnp.float32))

    full = lambda s: pl.BlockSpec(s, lambda i: (0, 0))
    out = pl.pallas_call(
        _tc_body,
        grid=(E1 // RMID,),
        in_specs=[
            pl.BlockSpec((RMID, D), lambda i: (i, 0)),
            pl.BlockSpec((RMID, D), lambda i: (i, 0)),
            full((D, D)), full((D, D)), full((1, D)),
            full((RMID // FANOUT, RMID)),
            full((B, D)), full((B, D)), full((D, D)), full((D, D)),
            full((1, D)), full((D, D)), full((D, D)), full((1, D)),
        ],
        out_specs=full((B, D)),
        out_shape=jax.ShapeDtypeStruct((B, D), jnp.float32),
        scratch_shapes=[pltpu.VMEM((B, D), jnp.float32)],
    )(h1_raw, m2_raw, P1, P2, c0v, pool, h0_raw, m1_raw, P3, P4, c0h,
      W1a, W1b, b12)
    return out


# speedup vs baseline: 6.7995x; 1.0912x over previous
import functools

import jax
import jax.numpy as jnp
from jax import lax
from jax.experimental import pallas as pl
from jax.experimental.pallas import tpu as pltpu
from jax.experimental.pallas import tpu_sc as plsc

D = 128
N_USERS = 50000
N_ITEMS = 50000
N_UFEAT = 3207
N_IFEAT = 2094
B = 1024
FANOUT = 16
FEAT = 8

NC, NS = 2, 16
NW = NC * NS

GROWS = 128
QCOLS = 32
NQ = D // QCOLS
NG = NW // NQ

U_PAD = 51200
AQ = 2
AG = NW // AQ
ACOLS = D // AQ
UG = U_PAD // AG
CU_A = 80
NCH_A = UG // CU_A

E2 = B * FANOUT * FANOUT
E2W = E2 // NW
CE_B = 128
NCH_B = E2W // CE_B
GPC_B = CE_B // FANOUT

E1 = B * FANOUT
EG_C = E1 // AG
CE_C = 64
NCH_C = EG_C // CE_C

RMID = 2048


def _worker_id():
    return lax.axis_index("s") * NC + lax.axis_index("c")


def _mean_rows(rows_v, acc_v, n_out, group, scale, out_base=0):
    def per_g(g, carry):
        base = g * group
        for dd in range(D // 16):
            sl = pl.ds(dd * 16, 16)
            t = [rows_v[base + 2 * k, sl] + rows_v[base + 2 * k + 1, sl]
                 for k in range(group // 2)]
            while len(t) > 1:
                t = [t[2 * k] + t[2 * k + 1] for k in range(len(t) // 2)]
            acc_v[out_base + g, sl] = t[0] * scale
        return carry
    lax.fori_loop(0, n_out, per_g, 0)


def _mean_rows_bf(rows_v, acc_v, n_out, group, scale, out_base=0):
    def per_g(g, carry):
        base = g * group
        for dd in range(D // 32):
            sl = pl.ds(dd * 32, 32)
            t = [rows_v[base + 2 * k, sl] + rows_v[base + 2 * k + 1, sl]
                 for k in range(group // 2)]
            while len(t) > 1:
                t = [t[2 * k] + t[2 * k + 1] for k in range(len(t) // 2)]
            acc_v[out_base + g, sl] = t[0] * scale
        return carry
    lax.fori_loop(0, n_out, per_g, 0)


def _fire_gather(table_hbm, idx_v, idx_off, rows_v, n_rows, sem):
    for h in range(n_rows // GROWS):
        pltpu.async_copy(
            table_hbm.at[idx_v.at[pl.ds(idx_off + h * GROWS, GROWS)]],
            rows_v.at[pl.ds(h * GROWS, GROWS)], sem)


def _wait_gather(table_hbm, idx_v, rows_v, n_rows, sem):
    for h in range(n_rows // GROWS):
        pltpu.make_async_copy(
            table_hbm.at[idx_v.at[pl.ds(0, GROWS)]],
            rows_v.at[pl.ds(h * GROWS, GROWS)], sem).wait()


@functools.lru_cache(maxsize=None)
def _build_sc_kernels():
    mesh = plsc.VectorSubcoreMesh(core_axis_name="c", subcore_axis_name="s")
    params = pltpu.CompilerParams(use_tc_tiling_on_sc=False,
                                  needs_layout_passes=False)

    @functools.partial(
        pl.kernel, mesh=mesh, compiler_params=params,
        out_type=jax.ShapeDtypeStruct((U_PAD, D), jnp.bfloat16),
        scratch_types=(
            [pltpu.VMEM((N_UFEAT, ACOLS), jnp.bfloat16)]
            + [pltpu.VMEM((CU_A * FEAT,), jnp.int32)] * 4
            + [pltpu.VMEM((CU_A, ACOLS), jnp.bfloat16)] * 4
            + [pltpu.SemaphoreType.DMA] * 8
        ),
    )
    def ufm_kernel(ufi_hbm, emb_hbm, out_hbm, table_v,
                   ix0, ix1, ix2, ix3, ot0, ot1, ot2, ot3,
                   is0, is1, is2, is3, ws0, ws1, ws2, ws3):
        ixs, ots = [ix0, ix1, ix2, ix3], [ot0, ot1, ot2, ot3]
        isems, wsems = [is0, is1, is2, is3], [ws0, ws1, ws2, ws3]
        wid = _worker_id()
        qid = wid % AQ
        gid = wid // AQ
        c0 = qid * ACOLS
        ub = gid * UG
        pltpu.sync_copy(emb_hbm.at[:, pl.ds(c0, ACOLS)], table_v)

        def fire_idx(i, idxv, sem):
            pltpu.async_copy(
                ufi_hbm.at[pl.ds((ub + i * CU_A) * FEAT, CU_A * FEAT)],
                idxv, sem)

        def compute(idxs, outv):
            def per_blk(pb, carry):
                vs = [idxs[pl.ds((pb * 4 + q) * 16, 16)] for q in range(4)]
                for q in range(4):
                    for half in range(2):
                        u = (pb * 4 + q) * 2 + half
                        r = [vs[q][half * FEAT + f] for f in range(FEAT)]
                        for s in range(ACOLS // 32):
                            sl = pl.ds(s * 32, 32)
                            t01 = table_v[r[0], sl] + table_v[r[1], sl]
                            t23 = table_v[r[2], sl] + table_v[r[3], sl]
                            t45 = table_v[r[4], sl] + table_v[r[5], sl]
                            t67 = table_v[r[6], sl] + table_v[r[7], sl]
                            outv[u, sl] = (((t01 + t23) + (t45 + t67))
                                           * (1.0 / FEAT))
                return carry
            lax.fori_loop(0, CU_A // 8, per_blk, 0)

        def quad(j, carry):
            i0 = 4 * j
            for r in range(4):
                i = i0 + r
                pltpu.make_async_copy(
                    ufi_hbm.at[pl.ds(ub * FEAT, CU_A * FEAT)],
                    ixs[r], isems[r]).wait()

                @pl.when(j > 0)
                def _():
                    pltpu.make_async_copy(
                        ots[r], out_hbm.at[pl.ds(ub, CU_A), pl.ds(c0, ACOLS)],
                        wsems[r]).wait()
                compute(ixs[r], ots[r])
                pltpu.async_copy(
                    ots[r],
                    out_hbm.at[pl.ds(ub + i * CU_A, CU_A), pl.ds(c0, ACOLS)],
                    wsems[r])

                @pl.when(i + 4 < NCH_A)
                def _():
                    fire_idx(i + 4, ixs[r], isems[r])
            return carry

        for r in range(4):
            fire_idx(r, ixs[r], isems[r])
        lax.fori_loop(0, NCH_A // 4, quad, 0)
        for r in range(4):
            pltpu.make_async_copy(
                ots[r], out_hbm.at[pl.ds(ub, CU_A), pl.ds(c0, ACOLS)],
                wsems[r]).wait()

    @functools.partial(
        pl.kernel, mesh=mesh, compiler_params=params,
        out_type=(jax.ShapeDtypeStruct((B, D), jnp.bfloat16),
                  jax.ShapeDtypeStruct((E1, D), jnp.bfloat16)),
        scratch_types=(
            [pltpu.VMEM((E2W,), jnp.int32)]
            + [pltpu.VMEM((CE_B, D), jnp.bfloat16)] * 4
            + [pltpu.VMEM((GPC_B, D), jnp.bfloat16)] * 4
            + [pltpu.VMEM((B // NW,), jnp.int32),
               pltpu.VMEM((B // NW, D), jnp.bfloat16)]
            + [pltpu.SemaphoreType.DMA] * 8
        ),
    )
    def l2_kernel(ufm_hbm, n0_hbm, n2_hbm, h0_hbm, m2_hbm, idx_v,
                  rw0, rw1, rw2, rw3, ac0, ac1, ac2, ac3, nbr0_v, rows0b_v,
                  gs0, gs1, gs2, gs3, ws0, ws1, ws2, ws3):
        rws, acs = [rw0, rw1, rw2, rw3], [ac0, ac1, ac2, ac3]
        gsems, wsems = [gs0, gs1, gs2, gs3], [ws0, ws1, ws2, ws3]
        wid = _worker_id()
        r0 = wid * (B // NW)
        pltpu.sync_copy(n0_hbm.at[pl.ds(r0, B // NW)], nbr0_v)
        h0sem = ws0
        pltpu.async_copy(ufm_hbm.at[nbr0_v], rows0b_v, h0sem)
        eb = wid * E2W
        gb = wid * (E2W // FANOUT)
        pltpu.sync_copy(n2_hbm.at[pl.ds(eb, E2W)], idx_v)

        def fire(i, rows, sem):
            pltpu.async_copy(
                ufm_hbm.at[idx_v.at[pl.ds(i * CE_B, CE_B)]], rows, sem)

        def quad(j, carry):
            i0 = 4 * j
            for r in range(4):
                i = i0 + r
                pltpu.make_async_copy(
                    ufm_hbm.at[idx_v.at[pl.ds(0, CE_B)]], rws[r],
                    gsems[r]).wait()

                @pl.when(j > 0)
                def _():
                    pltpu.make_async_copy(
                        acs[r], m2_hbm.at[pl.ds(gb, GPC_B)], wsems[r]).wait()
                _mean_rows_bf(rws[r], acs[r], GPC_B, FANOUT, 1.0 / FANOUT)
                pltpu.async_copy(
                    acs[r], m2_hbm.at[pl.ds(gb + i * GPC_B, GPC_B)], wsems[r])

                @pl.when(i + 4 < NCH_B)
                def _():
                    fire(i + 4, rws[r], gsems[r])
            return carry

        for r in range(4):
            fire(r, rws[r], gsems[r])
        pltpu.make_async_copy(ufm_hbm.at[nbr0_v], rows0b_v, h0sem).wait()
        pltpu.sync_copy(rows0b_v, h0_hbm.at[pl.ds(r0, B // NW)])
        lax.fori_loop(0, NCH_B // 4, quad, 0)
        for r in range(4):
            pltpu.make_async_copy(
                acs[r], m2_hbm.at[pl.ds(gb, GPC_B)], wsems[r]).wait()

    @functools.partial(
        pl.kernel, mesh=mesh, compiler_params=params,
        out_type=(jax.ShapeDtypeStruct((E1, D), jnp.bfloat16),
                  jax.ShapeDtypeStruct((B, D), jnp.bfloat16)),
        scratch_types=(
            [pltpu.VMEM((N_IFEAT, ACOLS), jnp.bfloat16),
             pltpu.VMEM((EG_C,), jnp.int32),
             pltpu.VMEM((EG_C, 2 * FEAT), jnp.int32)]
            + [pltpu.VMEM((CE_C, ACOLS), jnp.bfloat16)] * 4
            + [pltpu.VMEM((B // AG, ACOLS), jnp.bfloat16)]
            + [pltpu.SemaphoreType.DMA] * 5
        ),
    )
    def item_kernel(n1_hbm, ifi16_hbm, emb_hbm, h1_hbm, m1_hbm,
                    table_v, nbr_v, idx16_v, ot0, ot1, ot2, ot3, m1_v,
                    gsem, ws0, ws1, ws2, ws3):
        ots = [ot0, ot1, ot2, ot3]
        wsems = [ws0, ws1, ws2, ws3]
        wid = _worker_id()
        qid = wid % AQ
        gid = wid // AQ
        c0 = qid * ACOLS
        eb = gid * EG_C
        pltpu.sync_copy(n1_hbm.at[pl.ds(eb, EG_C)], nbr_v)
        for i in range(EG_C // GROWS):
            pltpu.async_copy(
                ifi16_hbm.at[nbr_v.at[pl.ds(i * GROWS, GROWS)]],
                idx16_v.at[pl.ds(i * GROWS, GROWS)], gsem)
        pltpu.sync_copy(emb_hbm.at[:, pl.ds(c0, ACOLS)], table_v)
        for i in range(EG_C // GROWS):
            pltpu.make_async_copy(
                ifi16_hbm.at[nbr_v.at[pl.ds(0, GROWS)]],
                idx16_v.at[pl.ds(i * GROWS, GROWS)], gsem).wait()

        def compute(i, outv):
            def per_blk(eb8, carry):
                vs = [idx16_v[i * CE_C + eb8 * 8 + k] for k in range(8)]
                for k in range(8):
                    e = eb8 * 8 + k
                    r = [vs[k][f] for f in range(FEAT)]
                    for s in range(ACOLS // 32):
                        sl = pl.ds(s * 32, 32)
                        t01 = table_v[r[0], sl] + table_v[r[1], sl]
                        t23 = table_v[r[2], sl] + table_v[r[3], sl]
                        t45 = table_v[r[4], sl] + table_v[r[5], sl]
                        t67 = table_v[r[6], sl] + table_v[r[7], sl]
                        outv[e, sl] = ((t01 + t23) + (t45 + t67)) * (1.0 / FEAT)
                return carry
            lax.fori_loop(0, CE_C // 8, per_blk, 0)
            def per_m(g, carry):
                for s in range(ACOLS // 32):
                    sl = pl.ds(s * 32, 32)
                    t = [outv[g * FANOUT + f, sl] + outv[g * FANOUT + f + 8, sl]
                         for f in range(8)]
                    acc = ((t[0] + t[1]) + (t[2] + t[3])) \
                        + ((t[4] + t[5]) + (t[6] + t[7]))
                    m1_v[i * (CE_C // FANOUT) + g, sl] = acc * (1.0 / FANOUT)
                return carry
            lax.fori_loop(0, CE_C // FANOUT, per_m, 0)

        def quad(j, carry):
            i0 = 4 * j
            for r in range(4):
                i = i0 + r

                @pl.when(j > 0)
                def _():
                    pltpu.make_async_copy(
                        ots[r], h1_hbm.at[pl.ds(eb, CE_C), pl.ds(c0, ACOLS)],
                        wsems[r]).wait()
                compute(i, ots[r])
                pltpu.async_copy(
                    ots[r],
                    h1_hbm.at[pl.ds(eb + i * CE_C, CE_C), pl.ds(c0, ACOLS)],
                    wsems[r])
            return carry

        lax.fori_loop(0, NCH_C // 4, quad, 0)
        for r in range(4):
            pltpu.make_async_copy(
                ots[r], h1_hbm.at[pl.ds(eb, CE_C), pl.ds(c0, ACOLS)],
                wsems[r]).wait()
        pltpu.sync_copy(
            m1_v, m1_hbm.at[pl.ds(gid * (B // AG), B // AG),
                            pl.ds(c0, ACOLS)])

    return ufm_kernel, l2_kernel, item_kernel


def _tc_body(h1r, m2r, P1, P2, c0v, pool, h0r, m1r, P3, P4, c0h,
             W1a, W1b, b1, out, mm1_acc):
    i = pl.program_id(0)
    nh1 = jnp.maximum(
        jnp.dot(h1r[...].astype(jnp.float32), P1[...],
                preferred_element_type=jnp.float32)
        + jnp.dot(m2r[...].astype(jnp.float32), P2[...],
                  preferred_element_type=jnp.float32)
        + c0v[...], 0.0)
    mm1_acc[pl.ds(i * (RMID // FANOUT), RMID // FANOUT), :] = jnp.dot(
        pool[...], nh1, preferred_element_type=jnp.float32)

    @pl.when(i == E1 // RMID - 1)
    def _():
        nh0 = jnp.maximum(
            jnp.dot(h0r[...].astype(jnp.float32), P3[...],
                    preferred_element_type=jnp.float32)
            + jnp.dot(m1r[...].astype(jnp.float32), P4[...],
                      preferred_element_type=jnp.float32)
            + c0h[...], 0.0)
        out[...] = (jnp.dot(nh0, W1a[...], preferred_element_type=jnp.float32)
                    + jnp.dot(mm1_acc[...], W1b[...],
                              preferred_element_type=jnp.float32)
                    + b1[...])


def kernel(neighbors_l0, neighbors_l1, neighbors_l2, offsets_l1, offsets_l2,
           user_feature_indices, user_feature_offsets, item_feature_indices,
           item_feature_offsets, user_feature_emb, item_feature_emb,
           user_proj_W, user_proj_b, item_proj_W, item_proj_b,
           w0_W, w0_b, w1_W, w1_b):
    n0 = neighbors_l0.astype(jnp.int32)
    n1 = neighbors_l1.astype(jnp.int32)
    n2 = neighbors_l2.astype(jnp.int32)
    ufi = user_feature_indices.astype(jnp.int32)
    ifi = item_feature_indices.astype(jnp.int32)

    ufi_pad = jnp.pad(ufi, (0, (U_PAD - N_USERS) * FEAT))
    ifi16 = jnp.tile(ifi.reshape(N_ITEMS, FEAT), (1, 2))

    ufm_kernel, l2_kernel, item_kernel = _build_sc_kernels()
    ufm = ufm_kernel(ufi_pad, user_feature_emb.astype(jnp.bfloat16))
    h0_raw, m2_raw = l2_kernel(ufm, n0, n2)
    h1_raw, m1_raw = item_kernel(n1, ifi16, item_feature_emb.astype(jnp.bfloat16))

    W0a, W0b = w0_W[:D], w0_W[D:]
    W1a, W1b = w1_W[:D], w1_W[D:]
    P1 = item_proj_W @ W0a
    P2 = user_proj_W @ W0b
    P3 = user_proj_W @ W0a
    P4 = item_proj_W @ W0b
    c0v = (item_proj_b @ W0a + user_proj_b @ W0b + w0_b)[None, :]
    c0h = (user_proj_b @ W0a + item_proj_b @ W0b + w0_b)[None, :]
    b12 = w1_b[None, :]
    pool = jnp.kron(jnp.eye(RMID // FANOUT, dtype=jnp.float32),
                    jnp.full((1, FANOUT), 1.0 / FANOUT, dtype=jnp.float32))

    full = lambda s: pl.BlockSpec(s, lambda i: (0, 0))
    out = pl.pallas_call(
        _tc_body,
        grid=(E1 // RMID,),
        in_specs=[
            pl.BlockSpec((RMID, D), lambda i: (i, 0)),
            pl.BlockSpec((RMID, D), lambda i: (i, 0)),
            full((D, D)), full((D, D)), full((1, D)),
            full((RMID // FANOUT, RMID)),
            full((B, D)), full((B, D)), full((D, D)), full((D, D)),
            full((1, D)), full((D, D)), full((D, D)), full((1, D)),
        ],
        out_specs=full((B, D)),
        out_shape=jax.ShapeDtypeStruct((B, D), jnp.float32),
        scratch_shapes=[pltpu.VMEM((B, D), jnp.float32)],
    )(h1_raw, m2_raw, P1, P2, c0v, pool, h0_raw, m1_raw, P3, P4, c0h,
      W1a, W1b, b12)
    return out
